# baseline (speedup 1.0000x reference)
import functools

import jax
import jax.numpy as jnp
from jax import lax
from jax.experimental import pallas as pl
from jax.experimental.pallas import tpu as pltpu
from jax.experimental.pallas import tpu_sc as plsc

D = 32
G = 128
K = 5


def kernel(indices, hashes, weight):
    B, L = indices.shape
    N = B * L
    info = plsc.get_sparse_core_info()
    NW = info.num_cores * info.num_subcores
    NS = info.num_subcores
    n_groups = N // (NW * G)
    n_rounds = n_groups // K
    n_w = n_groups * G

    idx3 = indices.reshape(NW, n_groups, G)
    h0col = hashes[:, 0]
    h1col = hashes[:, 1]

    @functools.partial(
        pl.kernel,
        mesh=plsc.VectorSubcoreMesh(core_axis_name="c", subcore_axis_name="s"),
        compiler_params=pltpu.CompilerParams(use_tc_tiling_on_sc=False),
        out_type=jax.ShapeDtypeStruct((N, D), jnp.float32),
        scratch_types=[
            pltpu.VMEM((n_groups, G), jnp.int32),
            pltpu.VMEM((2, K, G), jnp.int32),
            pltpu.VMEM((2, K, G), jnp.int32),
            pltpu.VMEM((K * G, D), jnp.float32),
            pltpu.SemaphoreType.DMA,
            pltpu.SemaphoreType.DMA,
        ],
    )
    def sc_kernel(idx_hbm, h0_hbm, h1_hbm, w_hbm, out_hbm,
                  idx_v, h0v, h1v, ebuf, sem_h, sem_e):
        wid = lax.axis_index("c") * NS + lax.axis_index("s")
        pltpu.sync_copy(idx_hbm.at[wid], idx_v)

        def fire_hash(r, p):
            for g in range(K):
                j = r * K + g
                pltpu.async_copy(h0_hbm.at[idx_v.at[j]], h0v.at[p, g], sem_h)
                pltpu.async_copy(h1_hbm.at[idx_v.at[j]], h1v.at[p, g], sem_h)

        def round_body(r, p):
            for _ in range(2 * K):
                pltpu.make_async_copy(
                    h0_hbm.at[idx_v.at[0]], h0v.at[0, 0], sem_h).wait()
            e_cps = []
            for g in range(K):
                e_cps.append(pltpu.async_copy(
                    w_hbm.at[h0v.at[p, g]],
                    ebuf.at[pl.ds(g * G, G)], sem_e))

            @pl.when(r + 1 < n_rounds)
            def _():
                fire_hash(r + 1, 1 - p)

            for cp in e_cps:
                cp.wait()
            a_cps = []
            for g in range(K):
                a_cps.append(pltpu.async_copy(
                    w_hbm.at[h1v.at[p, g]],
                    ebuf.at[pl.ds(g * G, G)], sem_e, add=True))
            for cp in a_cps:
                cp.wait()
            pltpu.sync_copy(ebuf, out_hbm.at[pl.ds(wid * n_w + r * (K * G),
                                                   K * G)])

        fire_hash(0, 0)

        def pair_body(t, carry):
            round_body(2 * t, 0)
            round_body(2 * t + 1, 1)
            return carry

        lax.fori_loop(0, n_rounds // 2, pair_body, 0)

    out = sc_kernel(idx3, h0col, h1col, weight)
    return out.reshape(B, L, D)


# speedup vs baseline: 1.9447x; 1.9447x over previous
import functools

import jax
import jax.numpy as jnp
from jax import lax
from jax.experimental import pallas as pl
from jax.experimental.pallas import tpu as pltpu
from jax.experimental.pallas import tpu_sc as plsc

D = 32
G = 128
K = 5
LANES = 16
TS = 129


def kernel(indices, hashes, weight):
    B, L = indices.shape
    info = plsc.get_sparse_core_info()
    NW = info.num_cores * info.num_subcores
    NS = info.num_subcores
    n_rounds = L // K
    BT = B // G

    idx_t = indices.T.reshape(L, B)
    h0col = hashes[:, 0]
    h1col = hashes[:, 1]

    @functools.partial(
        pl.kernel,
        mesh=plsc.VectorSubcoreMesh(core_axis_name="c", subcore_axis_name="s"),
        compiler_params=pltpu.CompilerParams(
            use_tc_tiling_on_sc=False, needs_layout_passes=False),
        out_type=jax.ShapeDtypeStruct((L, D // 8, BT, 8, G), jnp.float32),
        scratch_types=[
            pltpu.VMEM((L, G), jnp.int32),
            pltpu.VMEM((2, K, G), jnp.int32),
            pltpu.VMEM((2, K, G), jnp.int32),
            pltpu.VMEM((K * G, D), jnp.float32),
            pltpu.VMEM((K * G, D), jnp.float32),
            pltpu.VMEM((D, TS), jnp.float32),
            pltpu.SemaphoreType.DMA,
            pltpu.SemaphoreType.DMA,
            pltpu.SemaphoreType.DMA,
        ],
    )
    def sc_kernel(idx_hbm, h0_hbm, h1_hbm, w_hbm, out_hbm,
                  idx_v, h0v, h1v, ebuf0, ebuf1, tbuf, sem_h, sem_e, sem_o):
        wid = lax.axis_index("c") * NS + lax.axis_index("s")
        i_cps = [
            pltpu.async_copy(idx_hbm.at[l, pl.ds(wid * G, G)],
                             idx_v.at[l], sem_h)
            for l in range(L)
        ]
        for cp in i_cps:
            cp.wait()

        def fire_hash(r, p):
            for g in range(K):
                j = r * K + g
                pltpu.async_copy(h0_hbm.at[idx_v.at[j]], h0v.at[p, g], sem_h)
                pltpu.async_copy(h1_hbm.at[idx_v.at[j]], h1v.at[p, g], sem_h)

        def round_body(r, p):
            ebuf = ebuf0 if p == 0 else ebuf1
            for _ in range(2 * K):
                pltpu.make_async_copy(
                    h0_hbm.at[idx_v.at[0]], h0v.at[0, 0], sem_h).wait()
            e_cps = []
            for g in range(K):
                e_cps.append(pltpu.async_copy(
                    w_hbm.at[h0v.at[p, g]],
                    ebuf.at[pl.ds(g * G, G)], sem_e))

            @pl.when(r + 1 < n_rounds)
            def _():
                fire_hash(r + 1, 1 - p)

            for cp in e_cps:
                cp.wait()
            a_cps = []
            for g in range(K):
                a_cps.append(pltpu.async_copy(
                    w_hbm.at[h1v.at[p, g]],
                    ebuf.at[pl.ds(g * G, G)], sem_e, add=True))
            for cp in a_cps:
                cp.wait()

            for g in range(K):

                @pl.when((r > 0) | (g > 0))
                def _():
                    for dt in range(D // 8):
                        pltpu.make_async_copy(
                            tbuf.at[pl.ds(0, 8), pl.ds(0, G)],
                            out_hbm.at[0, 0, 0], sem_o).wait()

                pc = [c * LANES + lax.iota(jnp.int32, LANES)
                      for c in range(D // LANES)]

                def transpose_toks(it, carry):
                    for u in range(8):
                        tok = it * 8 + u
                        tokv = jnp.full((LANES,), tok, jnp.int32)
                        for c in range(D // LANES):
                            v = ebuf[g * G + tok, pl.ds(c * LANES, LANES)]
                            plsc.store_scatter(tbuf, [pc[c], tokv], v)
                    return carry

                lax.fori_loop(0, G // 8, transpose_toks, 0)
                for dt in range(D // 8):
                    pltpu.async_copy(
                        tbuf.at[pl.ds(dt * 8, 8), pl.ds(0, G)],
                        out_hbm.at[r * K + g, dt, wid], sem_o)

        fire_hash(0, 0)

        def pair_body(t, carry):
            round_body(2 * t, 0)
            round_body(2 * t + 1, 1)
            return carry

        lax.fori_loop(0, n_rounds // 2, pair_body, 0)
        for dt in range(D // 8):
            pltpu.make_async_copy(
                tbuf.at[pl.ds(0, 8), pl.ds(0, G)],
                out_hbm.at[0, 0, 0], sem_o).wait()

    out5 = sc_kernel(idx_t, h0col, h1col, weight)
    return out5.transpose(2, 4, 0, 1, 3).reshape(B, L, D)


# speedup vs baseline: 2.0971x; 1.0784x over previous
import functools

import jax
import jax.numpy as jnp
from jax import lax
from jax.experimental import pallas as pl
from jax.experimental.pallas import tpu as pltpu
from jax.experimental.pallas import tpu_sc as plsc

D = 32
G = 128
K = 5
LANES = 16
TS = 129


def kernel(indices, hashes, weight):
    B, L = indices.shape
    info = plsc.get_sparse_core_info()
    NW = info.num_cores * info.num_subcores
    NS = info.num_subcores
    n_rounds = L // K
    BT = B // G

    idx_t = indices.T.reshape(L, B)
    h0col = hashes[:, 0]
    h1col = hashes[:, 1]

    @functools.partial(
        pl.kernel,
        mesh=plsc.VectorSubcoreMesh(core_axis_name="c", subcore_axis_name="s"),
        compiler_params=pltpu.CompilerParams(
            use_tc_tiling_on_sc=False, needs_layout_passes=False),
        out_type=jax.ShapeDtypeStruct((L, D // 8, BT, 8, G), jnp.float32),
        scratch_types=[
            pltpu.VMEM((L, G), jnp.int32),
            pltpu.VMEM((2, K, G), jnp.int32),
            pltpu.VMEM((2, K, G), jnp.int32),
            pltpu.VMEM((K * G, D), jnp.float32),
            pltpu.VMEM((K * G, D), jnp.float32),
            pltpu.VMEM((D, TS), jnp.float32),
            pltpu.SemaphoreType.DMA,
            pltpu.SemaphoreType.DMA,
            pltpu.SemaphoreType.DMA,
        ],
    )
    def sc_kernel(idx_hbm, h0_hbm, h1_hbm, w_hbm, out_hbm,
                  idx_v, h0v, h1v, ebuf0, ebuf1, tbuf, sem_h, sem_e, sem_o):
        wid = lax.axis_index("c") * NS + lax.axis_index("s")
        i_cps = [
            pltpu.async_copy(idx_hbm.at[l, pl.ds(wid * G, G)],
                             idx_v.at[l], sem_h)
            for l in range(L)
        ]
        for cp in i_cps:
            cp.wait()

        def fire_hash(r, p):
            for g in range(K):
                j = r * K + g
                pltpu.async_copy(h0_hbm.at[idx_v.at[j]], h0v.at[p, g], sem_h)
                pltpu.async_copy(h1_hbm.at[idx_v.at[j]], h1v.at[p, g], sem_h)

        def drain_h():
            for _ in range(2 * K):
                pltpu.make_async_copy(
                    h0_hbm.at[idx_v.at[0]], h0v.at[0, 0], sem_h).wait()

        def fire_e0(r, p):
            ebuf = ebuf0 if p == 0 else ebuf1
            for g in range(K):
                pltpu.async_copy(
                    w_hbm.at[h0v.at[p, g]],
                    ebuf.at[pl.ds(g * G, G)], sem_e)

        def drain_e():
            for _ in range(K):
                pltpu.make_async_copy(
                    w_hbm.at[h0v.at[0, 0]],
                    (ebuf0 if True else ebuf0).at[pl.ds(0, G)], sem_e).wait()

        def round_body(r, p):
            ebuf = ebuf0 if p == 0 else ebuf1
            drain_e()
            for g in range(K):
                pltpu.async_copy(
                    w_hbm.at[h1v.at[p, g]],
                    ebuf.at[pl.ds(g * G, G)], sem_e, add=True)
            drain_e()

            @pl.when(r + 1 < n_rounds)
            def _():
                drain_h()
                fire_e0(r + 1, 1 - p)

            @pl.when(r + 2 < n_rounds)
            def _():
                fire_hash(r + 2, p)

            for g in range(K):

                @pl.when((r > 0) | (g > 0))
                def _():
                    for dt in range(D // 8):
                        pltpu.make_async_copy(
                            tbuf.at[pl.ds(0, 8), pl.ds(0, G)],
                            out_hbm.at[0, 0, 0], sem_o).wait()

                pc = [c * LANES + lax.iota(jnp.int32, LANES)
                      for c in range(D // LANES)]

                def transpose_toks(it, carry):
                    for u in range(8):
                        tok = it * 8 + u
                        tokv = jnp.full((LANES,), tok, jnp.int32)
                        for c in range(D // LANES):
                            v = ebuf[g * G + tok, pl.ds(c * LANES, LANES)]
                            plsc.store_scatter(tbuf, [pc[c], tokv], v)
                    return carry

                lax.fori_loop(0, G // 8, transpose_toks, 0)
                for dt in range(D // 8):
                    pltpu.async_copy(
                        tbuf.at[pl.ds(dt * 8, 8), pl.ds(0, G)],
                        out_hbm.at[r * K + g, dt, wid], sem_o)

        fire_hash(0, 0)
        drain_h()
        fire_e0(0, 0)
        fire_hash(1, 1)

        def pair_body(t, carry):
            round_body(2 * t, 0)
            round_body(2 * t + 1, 1)
            return carry

        lax.fori_loop(0, n_rounds // 2, pair_body, 0)
        for dt in range(D // 8):
            pltpu.make_async_copy(
                tbuf.at[pl.ds(0, 8), pl.ds(0, G)],
                out_hbm.at[0, 0, 0], sem_o).wait()

    out5 = sc_kernel(idx_t, h0col, h1col, weight)
    return out5.transpose(2, 4, 0, 1, 3).reshape(B, L, D)
